# baseline (speedup 1.0000x reference)
import functools

import jax
import jax.numpy as jnp
from jax import lax
from jax.experimental import pallas as pl
from jax.experimental.pallas import tpu as pltpu
from jax.experimental.pallas import tpu_sc as plsc

TOPK = 16


def _topk_body(q_ref, k_ref, vals_ref, idx_ref, sim_ref, rv_ref, ri_ref,
               bv_ref, bi_ref, *, kb_size, num_blocks, num_keys):
    kb = pl.program_id(0)
    big = jnp.int32(2**31 - 1)

    @pl.when(kb == 0)
    def _init():
        rv_ref[...] = jnp.full(rv_ref.shape, -jnp.inf, jnp.float32)
        ri_ref[...] = jnp.full(ri_ref.shape, big, jnp.int32)

    qn = q_ref[...]
    kn = k_ref[...]
    mm = lax.dot_general(qn, kn, (((1,), (1,)), ((), ())),
                         preferred_element_type=jnp.float32,
                         precision=lax.Precision.DEFAULT)
    gcol = kb * kb_size + lax.broadcasted_iota(jnp.int32, mm.shape, 1)
    mm = jnp.where(gcol < num_keys, mm, -jnp.inf)
    sim_ref[...] = mm

    rmin = rv_ref[:, TOPK - 1:TOPK]
    cnt = jnp.sum((mm > rmin).astype(jnp.int32), axis=1, keepdims=True)
    c = jnp.minimum(jnp.max(cnt), TOPK)

    bv_ref[...] = jnp.full(bv_ref.shape, -jnp.inf, jnp.float32)
    bi_ref[...] = jnp.full(bi_ref.shape, big, jnp.int32)
    lane16 = lax.broadcasted_iota(jnp.int32, (bv_ref.shape[0], TOPK), 1)

    for r in range(TOPK):
        @pl.when(r < c)
        def _round(r=r):
            s = sim_ref[...]
            m = jnp.max(s, axis=1, keepdims=True)
            sel = jnp.min(jnp.where(s == m, gcol, big), axis=1, keepdims=True)
            bv_ref[...] = jnp.where(lane16 == r, m, bv_ref[...])
            bi_ref[...] = jnp.where(lane16 == r, sel, bi_ref[...])

            @pl.when(r + 1 < c)
            def _mask():
                sim_ref[...] = jnp.where(gcol == sel, -jnp.inf, s)

    @pl.when(c > 0)
    def _merge():
        cv = jnp.concatenate([rv_ref[...], bv_ref[...]], axis=1)
        ci = jnp.concatenate([ri_ref[...], bi_ref[...]], axis=1)
        new_vals = []
        new_idx = []
        for _ in range(TOPK):
            m = jnp.max(cv, axis=1, keepdims=True)
            sel = jnp.min(jnp.where(cv == m, ci, big), axis=1, keepdims=True)
            new_vals.append(m)
            new_idx.append(sel)
            cv = jnp.where(ci == sel, -jnp.inf, cv)
        rv_ref[...] = jnp.concatenate(new_vals, axis=1)
        ri_ref[...] = jnp.concatenate(new_idx, axis=1)

    @pl.when(kb == num_blocks - 1)
    def _out():
        vals_ref[...] = rv_ref[...]
        idx_ref[...] = ri_ref[...]


def _topk_pallas(queries, keys_padded, num_keys, kb_size):
    q, d = queries.shape
    nb = (keys_padded.shape[0] + kb_size - 1) // kb_size
    body = functools.partial(_topk_body, kb_size=kb_size, num_blocks=nb,
                             num_keys=num_keys)
    return pl.pallas_call(
        body,
        grid=(nb,),
        in_specs=[
            pl.BlockSpec((q, d), lambda kb: (0, 0)),
            pl.BlockSpec((kb_size, d), lambda kb: (kb, 0)),
        ],
        out_specs=[
            pl.BlockSpec((q, TOPK), lambda kb: (0, 0)),
            pl.BlockSpec((q, TOPK), lambda kb: (0, 0)),
        ],
        out_shape=[
            jax.ShapeDtypeStruct((q, TOPK), jnp.float32),
            jax.ShapeDtypeStruct((q, TOPK), jnp.int32),
        ],
        scratch_shapes=[
            pltpu.VMEM((q, kb_size), jnp.float32),
            pltpu.VMEM((q, TOPK), jnp.float32),
            pltpu.VMEM((q, TOPK), jnp.int32),
            pltpu.VMEM((q, TOPK), jnp.float32),
            pltpu.VMEM((q, TOPK), jnp.int32),
        ],
    )(queries, keys_padded)


def _gather_mean_sc(keys, idx_flat, q, d):
    info = plsc.get_sparse_core_info()
    nw = info.num_cores * info.num_subcores
    nq = q // nw
    mesh = plsc.VectorSubcoreMesh(core_axis_name="c", subcore_axis_name="s")

    @functools.partial(
        pl.kernel,
        mesh=mesh,
        out_type=jax.ShapeDtypeStruct((q, d), jnp.float32),
        scratch_types=[
            pltpu.VMEM((nq * TOPK,), jnp.int32),
            pltpu.VMEM((TOPK, d), jnp.float32),
            pltpu.VMEM((nq, d), jnp.float32),
            pltpu.SemaphoreType.DMA,
        ],
    )
    def gather_kernel(keys_hbm, idx_hbm, out_hbm, idx_v, rows_v, out_v, sem):
        wid = lax.axis_index("s") * info.num_cores + lax.axis_index("c")
        base = wid * (nq * TOPK)
        pltpu.sync_copy(idx_hbm.at[pl.ds(base, nq * TOPK)], idx_v)

        def body(qi, carry):
            iv = idx_v[pl.ds(qi * TOPK, TOPK)]
            pltpu.async_copy(keys_hbm.at[iv], rows_v, sem).wait()
            for c in range(d // 16):
                a = rows_v[0, pl.ds(c * 16, 16)]
                for r in range(1, TOPK):
                    a = a + rows_v[r, pl.ds(c * 16, 16)]
                out_v[qi, pl.ds(c * 16, 16)] = a * (1.0 / TOPK)
            return carry

        lax.fori_loop(0, nq, body, jnp.int32(0))
        pltpu.sync_copy(out_v, out_hbm.at[pl.ds(wid * nq, nq), :])

    return gather_kernel(keys, idx_flat)


def kernel(queries, keys, k):
    del k
    q, d = queries.shape
    num_keys = keys.shape[0]
    kb_size = 2048
    kp = ((num_keys + kb_size - 1) // kb_size) * kb_size
    qn = queries / jnp.clip(
        jnp.linalg.norm(queries, axis=-1, keepdims=True), 1e-12)
    kn = keys / jnp.clip(jnp.linalg.norm(keys, axis=-1, keepdims=True), 1e-12)
    del kp
    vals, idx = _topk_pallas(qn, kn, num_keys, kb_size)
    proto = _gather_mean_sc(keys, idx.reshape(-1), q, d)
    return vals, idx, proto


# speedup vs baseline: 1.6356x; 1.6356x over previous
import functools

import jax
import jax.numpy as jnp
from jax import lax
from jax.experimental import pallas as pl
from jax.experimental.pallas import tpu as pltpu
from jax.experimental.pallas import tpu_sc as plsc

TOPK = 16


def _topk_body(q_ref, k_ref, vals_ref, idx_ref, sim_ref, rv_ref, ri_ref,
               bv_ref, bi_ref, *, kb_size, num_blocks, num_keys):
    kb = pl.program_id(0)
    big = jnp.int32(2**31 - 1)

    @pl.when(kb == 0)
    def _init():
        rv_ref[...] = jnp.full(rv_ref.shape, -jnp.inf, jnp.float32)
        ri_ref[...] = jnp.full(ri_ref.shape, big, jnp.int32)

    qn = q_ref[...]
    kn = k_ref[...]
    mm = lax.dot_general(qn, kn, (((1,), (1,)), ((), ())),
                         preferred_element_type=jnp.float32,
                         precision=lax.Precision.DEFAULT)
    gcol = kb * kb_size + lax.broadcasted_iota(jnp.int32, mm.shape, 1)
    mm = jnp.where(gcol < num_keys, mm, -jnp.inf)
    sim_ref[...] = mm

    rmin = rv_ref[:, TOPK - 1:TOPK]
    cnt = jnp.sum((mm > rmin).astype(jnp.int32), axis=1, keepdims=True)
    c = jnp.minimum(jnp.max(cnt), TOPK)

    bv_ref[...] = jnp.full(bv_ref.shape, -jnp.inf, jnp.float32)
    bi_ref[...] = jnp.full(bi_ref.shape, big, jnp.int32)
    lane16 = lax.broadcasted_iota(jnp.int32, (bv_ref.shape[0], TOPK), 1)

    for r in range(TOPK):
        @pl.when(r < c)
        def _round(r=r):
            s = sim_ref[...]
            m = jnp.max(s, axis=1, keepdims=True)
            sel = jnp.min(jnp.where(s == m, gcol, big), axis=1, keepdims=True)
            bv_ref[...] = jnp.where(lane16 == r, m, bv_ref[...])
            bi_ref[...] = jnp.where(lane16 == r, sel, bi_ref[...])

            @pl.when(r + 1 < c)
            def _mask():
                sim_ref[...] = jnp.where(gcol == sel, -jnp.inf, s)

    @pl.when(c > 0)
    def _merge():
        cv = jnp.concatenate([rv_ref[...], bv_ref[...]], axis=1)
        ci = jnp.concatenate([ri_ref[...], bi_ref[...]], axis=1)
        new_vals = []
        new_idx = []
        for _ in range(TOPK):
            m = jnp.max(cv, axis=1, keepdims=True)
            sel = jnp.min(jnp.where(cv == m, ci, big), axis=1, keepdims=True)
            new_vals.append(m)
            new_idx.append(sel)
            cv = jnp.where(ci == sel, -jnp.inf, cv)
        rv_ref[...] = jnp.concatenate(new_vals, axis=1)
        ri_ref[...] = jnp.concatenate(new_idx, axis=1)

    @pl.when(kb == num_blocks - 1)
    def _out():
        vals_ref[...] = rv_ref[...]
        idx_ref[...] = ri_ref[...]


def _topk_pallas(queries, keys_padded, num_keys, kb_size):
    q, d = queries.shape
    nb = (keys_padded.shape[0] + kb_size - 1) // kb_size
    body = functools.partial(_topk_body, kb_size=kb_size, num_blocks=nb,
                             num_keys=num_keys)
    return pl.pallas_call(
        body,
        grid=(nb,),
        in_specs=[
            pl.BlockSpec((q, d), lambda kb: (0, 0)),
            pl.BlockSpec((kb_size, d), lambda kb: (kb, 0)),
        ],
        out_specs=[
            pl.BlockSpec((q, TOPK), lambda kb: (0, 0)),
            pl.BlockSpec((q, TOPK), lambda kb: (0, 0)),
        ],
        out_shape=[
            jax.ShapeDtypeStruct((q, TOPK), jnp.float32),
            jax.ShapeDtypeStruct((q, TOPK), jnp.int32),
        ],
        scratch_shapes=[
            pltpu.VMEM((q, kb_size), jnp.float32),
            pltpu.VMEM((q, TOPK), jnp.float32),
            pltpu.VMEM((q, TOPK), jnp.int32),
            pltpu.VMEM((q, TOPK), jnp.float32),
            pltpu.VMEM((q, TOPK), jnp.int32),
        ],
    )(queries, keys_padded)


def _gather_mean_sc(keys, idx_flat, q, d):
    info = plsc.get_sparse_core_info()
    nw = info.num_cores * info.num_subcores
    nq = q // nw
    mesh = plsc.VectorSubcoreMesh(core_axis_name="c", subcore_axis_name="s")

    batch = 4

    @functools.partial(
        pl.kernel,
        mesh=mesh,
        out_type=jax.ShapeDtypeStruct((q, d), jnp.float32),
        scratch_types=[
            pltpu.VMEM((nq * TOPK,), jnp.int32),
            pltpu.VMEM((batch * TOPK, d), jnp.float32),
            pltpu.VMEM((nq, d), jnp.float32),
            pltpu.SemaphoreType.DMA,
        ],
    )
    def gather_kernel(keys_hbm, idx_hbm, out_hbm, idx_v, rows_v, out_v, sem):
        wid = lax.axis_index("s") * info.num_cores + lax.axis_index("c")
        base = wid * (nq * TOPK)
        pltpu.sync_copy(idx_hbm.at[pl.ds(base, nq * TOPK)], idx_v)

        def body(bi, carry):
            iref = idx_v.at[pl.ds(bi * (batch * TOPK), batch * TOPK)]
            pltpu.async_copy(keys_hbm.at[iref], rows_v, sem).wait()
            for j in range(batch):
                for c in range(d // 16):
                    a = rows_v[j * TOPK, pl.ds(c * 16, 16)]
                    for r in range(1, TOPK):
                        a = a + rows_v[j * TOPK + r, pl.ds(c * 16, 16)]
                    out_v[bi * batch + j, pl.ds(c * 16, 16)] = a * (1.0 / TOPK)
            return carry

        lax.fori_loop(0, nq // batch, body, jnp.int32(0))
        pltpu.sync_copy(out_v, out_hbm.at[pl.ds(wid * nq, nq), :])

    return gather_kernel(keys, idx_flat)


def kernel(queries, keys, k):
    del k
    q, d = queries.shape
    num_keys = keys.shape[0]
    kb_size = 1024
    kp = ((num_keys + kb_size - 1) // kb_size) * kb_size
    qn = queries / jnp.clip(
        jnp.linalg.norm(queries, axis=-1, keepdims=True), 1e-12)
    kn = keys / jnp.clip(jnp.linalg.norm(keys, axis=-1, keepdims=True), 1e-12)
    del kp
    vals, idx = _topk_pallas(qn, kn, num_keys, kb_size)
    proto = _gather_mean_sc(keys, idx.reshape(-1), q, d)
    return vals, idx, proto


# speedup vs baseline: 1.6548x; 1.0118x over previous
import functools

import jax
import jax.numpy as jnp
from jax import lax
from jax.experimental import pallas as pl
from jax.experimental.pallas import tpu as pltpu
from jax.experimental.pallas import tpu_sc as plsc

TOPK = 16


def _topk_body(q_ref, k_ref, vals_ref, idx_ref, sim_ref, rv_ref, ri_ref,
               bv_ref, bi_ref, *, kb_size, num_blocks, num_keys):
    kb = pl.program_id(0)
    big = jnp.int32(2**31 - 1)

    @pl.when(kb == 0)
    def _init():
        rv_ref[...] = jnp.full(rv_ref.shape, -jnp.inf, jnp.float32)
        ri_ref[...] = jnp.full(ri_ref.shape, big, jnp.int32)

    qn = q_ref[...]
    kn = k_ref[...]
    mm = lax.dot_general(qn, kn, (((1,), (1,)), ((), ())),
                         preferred_element_type=jnp.float32,
                         precision=lax.Precision.DEFAULT)
    gcol = kb * kb_size + lax.broadcasted_iota(jnp.int32, mm.shape, 1)
    mm = jnp.where(gcol < num_keys, mm, -jnp.inf)
    sim_ref[...] = mm

    rmin = rv_ref[:, TOPK - 1:TOPK]
    cnt = jnp.sum((mm > rmin).astype(jnp.int32), axis=1, keepdims=True)
    c = jnp.minimum(jnp.max(cnt), TOPK)

    bv_ref[...] = jnp.full(bv_ref.shape, -jnp.inf, jnp.float32)
    bi_ref[...] = jnp.full(bi_ref.shape, big, jnp.int32)
    lane16 = lax.broadcasted_iota(jnp.int32, (bv_ref.shape[0], TOPK), 1)

    for r in range(TOPK):
        @pl.when(r < c)
        def _round(r=r):
            s = sim_ref[...]
            m = jnp.max(s, axis=1, keepdims=True)
            sel = jnp.min(jnp.where(s == m, gcol, big), axis=1, keepdims=True)
            bv_ref[...] = jnp.where(lane16 == r, m, bv_ref[...])
            bi_ref[...] = jnp.where(lane16 == r, sel, bi_ref[...])

            @pl.when(r + 1 < c)
            def _mask():
                sim_ref[...] = jnp.where(gcol == sel, -jnp.inf, s)

    @pl.when(c > 0)
    def _merge():
        cv = jnp.concatenate([rv_ref[...], bv_ref[...]], axis=1)
        ci = jnp.concatenate([ri_ref[...], bi_ref[...]], axis=1)
        new_vals = []
        new_idx = []
        for _ in range(TOPK):
            m = jnp.max(cv, axis=1, keepdims=True)
            sel = jnp.min(jnp.where(cv == m, ci, big), axis=1, keepdims=True)
            new_vals.append(m)
            new_idx.append(sel)
            cv = jnp.where(ci == sel, -jnp.inf, cv)
        rv_ref[...] = jnp.concatenate(new_vals, axis=1)
        ri_ref[...] = jnp.concatenate(new_idx, axis=1)

    @pl.when(kb == num_blocks - 1)
    def _out():
        vals_ref[...] = rv_ref[...]
        idx_ref[...] = ri_ref[...]


def _topk_pallas(queries, keys_padded, num_keys, kb_size):
    q, d = queries.shape
    nb = (keys_padded.shape[0] + kb_size - 1) // kb_size
    body = functools.partial(_topk_body, kb_size=kb_size, num_blocks=nb,
                             num_keys=num_keys)
    return pl.pallas_call(
        body,
        grid=(nb,),
        in_specs=[
            pl.BlockSpec((q, d), lambda kb: (0, 0)),
            pl.BlockSpec((kb_size, d), lambda kb: (kb, 0)),
        ],
        out_specs=[
            pl.BlockSpec((q, TOPK), lambda kb: (0, 0)),
            pl.BlockSpec((q, TOPK), lambda kb: (0, 0)),
        ],
        out_shape=[
            jax.ShapeDtypeStruct((q, TOPK), jnp.float32),
            jax.ShapeDtypeStruct((q, TOPK), jnp.int32),
        ],
        scratch_shapes=[
            pltpu.VMEM((q, kb_size), jnp.float32),
            pltpu.VMEM((q, TOPK), jnp.float32),
            pltpu.VMEM((q, TOPK), jnp.int32),
            pltpu.VMEM((q, TOPK), jnp.float32),
            pltpu.VMEM((q, TOPK), jnp.int32),
        ],
    )(queries, keys_padded)


def _gather_mean_sc(keys, idx_flat, q, d):
    info = plsc.get_sparse_core_info()
    nw = info.num_cores * info.num_subcores
    nq = q // nw
    mesh = plsc.VectorSubcoreMesh(core_axis_name="c", subcore_axis_name="s")

    batch = 1

    @functools.partial(
        pl.kernel,
        mesh=mesh,
        out_type=jax.ShapeDtypeStruct((q, d), jnp.float32),
        scratch_types=[
            pltpu.VMEM((nq * TOPK,), jnp.int32),
            pltpu.VMEM((batch * TOPK, d), jnp.float32),
            pltpu.VMEM((nq, d), jnp.float32),
            pltpu.SemaphoreType.DMA,
        ],
    )
    def gather_kernel(keys_hbm, idx_hbm, out_hbm, idx_v, rows_v, out_v, sem):
        wid = lax.axis_index("s") * info.num_cores + lax.axis_index("c")
        base = wid * (nq * TOPK)
        pltpu.sync_copy(idx_hbm.at[pl.ds(base, nq * TOPK)], idx_v)

        def body(bi, carry):
            iref = idx_v.at[pl.ds(bi * (batch * TOPK), batch * TOPK)]
            pltpu.async_copy(keys_hbm.at[iref], rows_v, sem).wait()
            for j in range(batch):
                for c in range(d // 16):
                    a = rows_v[j * TOPK, pl.ds(c * 16, 16)]
                    for r in range(1, TOPK):
                        a = a + rows_v[j * TOPK + r, pl.ds(c * 16, 16)]
                    out_v[bi * batch + j, pl.ds(c * 16, 16)] = a * (1.0 / TOPK)
            return carry

        lax.fori_loop(0, nq // batch, body, jnp.int32(0))
        pltpu.sync_copy(out_v, out_hbm.at[pl.ds(wid * nq, nq), :])

    return gather_kernel(keys, idx_flat)


def kernel(queries, keys, k):
    del k
    q, d = queries.shape
    num_keys = keys.shape[0]
    kb_size = 1024
    kp = ((num_keys + kb_size - 1) // kb_size) * kb_size
    qn = queries / jnp.clip(
        jnp.linalg.norm(queries, axis=-1, keepdims=True), 1e-12)
    kn = keys / jnp.clip(jnp.linalg.norm(keys, axis=-1, keepdims=True), 1e-12)
    del kp
    vals, idx = _topk_pallas(qn, kn, num_keys, kb_size)
    proto = _gather_mean_sc(keys, idx.reshape(-1), q, d)
    return vals, idx, proto
